# Initial kernel scaffold; baseline (speedup 1.0000x reference)
import jax
import jax.numpy as jnp
from jax.experimental import pallas as pl


def kernel(x, cell_line, film):
    raise NotImplementedError("write your pallas kernel here")


# speedup vs baseline: 1.1999x; 1.1999x over previous
import functools

import jax
import jax.numpy as jnp
from jax import lax
from jax.experimental import pallas as pl
from jax.experimental.pallas import tpu as pltpu
from jax.experimental.pallas import tpu_sc as plsc

BATCH = 16384
D = 64
NC = 2
NS = 16
L = 16
NW = NC * NS
BPW = BATCH // NW
CHUNK = 256
NCHUNK = BPW // CHUNK

_mesh = plsc.VectorSubcoreMesh(core_axis_name="c", subcore_axis_name="s")


@functools.partial(
    pl.kernel,
    mesh=_mesh,
    out_type=jax.ShapeDtypeStruct((BATCH, D), jnp.float32),
    scratch_types=[
        pltpu.VMEM((CHUNK,), jnp.int32),
        pltpu.VMEM((CHUNK, 2 * D), jnp.float32),
        pltpu.VMEM((CHUNK, D), jnp.float32),
        pltpu.SemaphoreType.DMA,
    ],
)
def _film(x_hbm, idx_hbm, film_hbm, out_hbm, idx_v, rows_v, x_v, sem):
    wid = lax.axis_index("s") * NC + lax.axis_index("c")
    base = wid * BPW
    for c in range(NCHUNK):
        off = base + c * CHUNK
        pltpu.sync_copy(idx_hbm.at[pl.ds(off, CHUNK)], idx_v)
        gather = pltpu.async_copy(film_hbm.at[idx_v], rows_v, sem)
        pltpu.sync_copy(x_hbm.at[pl.ds(off, CHUNK)], x_v)
        gather.wait()

        def body(r, carry):
            for j in range(D // L):
                sl = pl.ds(j * L, L)
                g = rows_v[r, sl]
                b = rows_v[r, pl.ds(D + j * L, L)]
                x_v[r, sl] = g * x_v[r, sl] + b
            return carry

        lax.fori_loop(0, CHUNK, body, 0)
        pltpu.sync_copy(x_v, out_hbm.at[pl.ds(off, CHUNK)])


def kernel(x, cell_line, film):
    out = _film(x, cell_line, film)
    return (out, cell_line)


# speedup vs baseline: 1.2358x; 1.0300x over previous
import functools

import jax
import jax.numpy as jnp
from jax import lax
from jax.experimental import pallas as pl
from jax.experimental.pallas import tpu as pltpu
from jax.experimental.pallas import tpu_sc as plsc

BATCH = 16384
D = 64
NC = 2
NS = 16
L = 16
NW = NC * NS
BPW = BATCH // NW
CHUNK = 128
NCHUNK = BPW // CHUNK

_mesh = plsc.VectorSubcoreMesh(core_axis_name="c", subcore_axis_name="s")


@functools.partial(
    pl.kernel,
    mesh=_mesh,
    out_type=jax.ShapeDtypeStruct((BATCH, D), jnp.float32),
    scratch_types=[
        pltpu.VMEM((NCHUNK, CHUNK), jnp.int32),
        pltpu.VMEM((CHUNK, 2 * D), jnp.float32),
        pltpu.VMEM((CHUNK, 2 * D), jnp.float32),
        pltpu.VMEM((CHUNK, D), jnp.float32),
        pltpu.VMEM((CHUNK, D), jnp.float32),
        pltpu.SemaphoreType.DMA,
        pltpu.SemaphoreType.DMA,
        pltpu.SemaphoreType.DMA,
        pltpu.SemaphoreType.DMA,
        pltpu.SemaphoreType.DMA,
        pltpu.SemaphoreType.DMA,
    ],
)
def _film(x_hbm, idx_hbm, film_hbm, out_hbm,
          idx_v, rows0, rows1, xb0, xb1,
          gs0, gs1, xs0, xs1, os0, os1):
    rows = (rows0, rows1)
    xb = (xb0, xb1)
    gsem = (gs0, gs1)
    xsem = (xs0, xs1)
    osem = (os0, os1)

    wid = lax.axis_index("s") * NC + lax.axis_index("c")
    base = wid * BPW
    pltpu.sync_copy(idx_hbm.at[wid], idx_v)

    gathers = [None, None]
    xcopies = [None, None]
    ostores = [None, None]

    def start(c):
        b = c % 2
        gathers[b] = pltpu.async_copy(film_hbm.at[idx_v.at[c]], rows[b], gsem[b])
        xcopies[b] = pltpu.async_copy(
            x_hbm.at[pl.ds(base + c * CHUNK, CHUNK)], xb[b], xsem[b])

    start(0)
    for c in range(NCHUNK):
        b = c % 2
        if c + 1 < NCHUNK:
            nb = (c + 1) % 2
            if c >= 1:
                ostores[nb].wait()
            start(c + 1)
        gathers[b].wait()
        xcopies[b].wait()

        @plsc.parallel_loop(0, CHUNK, unroll=8)
        def body(r):
            for j in range(D // L):
                sl = pl.ds(j * L, L)
                xb[b][r, sl] = rows[b][r, sl] * xb[b][r, sl] \
                    + rows[b][r, pl.ds(D + j * L, L)]

        ostores[b] = pltpu.async_copy(
            xb[b], out_hbm.at[pl.ds(base + c * CHUNK, CHUNK)], osem[b])

    ostores[0].wait()
    ostores[1].wait()


def kernel(x, cell_line, film):
    idx = cell_line.reshape(NW, NCHUNK, CHUNK)
    out = _film(x, idx, film)
    return (out, cell_line)
